# baseline (speedup 1.0000x reference)
import functools

import jax
import jax.numpy as jnp
from jax import lax
from jax.experimental import pallas as pl
from jax.experimental.pallas import tpu as pltpu
from jax.experimental.pallas import tpu_sc as plsc

VOCAB = 1_000_000
SEQ = 200
D = 64
BATCH = 4096

NC, NS = 2, 16
NW = NC * NS
B_TOTAL = BATCH * SEQ
B_PER_W = B_TOTAL // NW
IBLK = 128
CHUNK = 640
KBLK = CHUNK // IBLK
N_CHUNKS = B_PER_W // CHUNK
POS_TILE = 3200


def _emb_kernel(idx_hbm, table_hbm, pos_hbm, out_hbm, idx_v, rows_v, sem):
    wid = lax.axis_index("s") * NC + lax.axis_index("c")

    def chunk_body(g, _):
        row0 = wid * B_PER_W + g * CHUNK
        pltpu.sync_copy(idx_hbm.at[pl.ds(row0, CHUNK)], idx_v)
        pos0 = (g % (POS_TILE // CHUNK)) * CHUNK
        pltpu.sync_copy(pos_hbm.at[pl.ds(pos0, CHUNK)], rows_v)
        descs = [
            pltpu.async_copy(
                table_hbm.at[idx_v.at[pl.ds(j * IBLK, IBLK)]],
                rows_v.at[pl.ds(j * IBLK, IBLK)],
                sem,
                add=True,
            )
            for j in range(KBLK)
        ]
        for d in descs:
            d.wait()
        pltpu.sync_copy(rows_v, out_hbm.at[pl.ds(row0, CHUNK)])
        return _

    lax.fori_loop(0, N_CHUNKS, chunk_body, None)


@jax.jit
def _embed(idx_blocks, token_table, pos_tiled):
    mesh = plsc.VectorSubcoreMesh(
        core_axis_name="c", subcore_axis_name="s", num_cores=NC, num_subcores=NS
    )
    fn = pl.kernel(
        _emb_kernel,
        out_type=jax.ShapeDtypeStruct((B_TOTAL, D), jnp.float32),
        mesh=mesh,
        scratch_types=[
            pltpu.VMEM((CHUNK,), jnp.int32),
            pltpu.VMEM((CHUNK, D), jnp.float32),
            pltpu.SemaphoreType.DMA,
        ],
        compiler_params=pltpu.CompilerParams(use_tc_tiling_on_sc=False),
    )
    return fn(idx_blocks, token_table, pos_tiled)


def kernel(inputs, token_table, pos_table):
    idx_blocks = inputs.astype(jnp.int32).reshape(B_TOTAL)
    pos_tiled = jnp.tile(pos_table.astype(jnp.float32), (POS_TILE // SEQ, 1))
    out = _embed(idx_blocks, token_table.astype(jnp.float32), pos_tiled)
    return out.reshape(BATCH, SEQ, D)


# speedup vs baseline: 1.0630x; 1.0630x over previous
import jax
import jax.numpy as jnp
from jax import lax
from jax.experimental import pallas as pl
from jax.experimental.pallas import tpu as pltpu
from jax.experimental.pallas import tpu_sc as plsc

VOCAB = 1_000_000
SEQ = 200
D = 64
BATCH = 4096

NC, NS = 2, 16
NW = NC * NS
B_TOTAL = BATCH * SEQ
B_PER_W = B_TOTAL // NW
IBLK = 128
CHUNK = 640
KBLK = CHUNK // IBLK
N_CHUNKS = B_PER_W // CHUNK
POS_TILE = 3200
NFILL = POS_TILE // CHUNK


def _emb_kernel(idx_hbm, table_hbm, pos_hbm, out_hbm, idx_v, rows_v,
                in_s0, in_s1, g_s0, g_s1, o_s0, o_s1):
    in_sem = (in_s0, in_s1)
    g_sem = (g_s0, g_s1)
    out_sem = (o_s0, o_s1)
    wid = lax.axis_index("s") * NC + lax.axis_index("c")
    base = wid * B_PER_W

    def in_descs(g, s):
        row0 = base + g * CHUNK
        pos0 = lax.rem(g, NFILL) * CHUNK
        return (
            pltpu.make_async_copy(
                idx_hbm.at[pl.ds(row0, CHUNK)], idx_v.at[s], in_sem[s]),
            pltpu.make_async_copy(
                pos_hbm.at[pl.ds(pos0, CHUNK)], rows_v.at[s], in_sem[s]),
        )

    def start_in(g, s):
        for d in in_descs(g, s):
            d.start()

    def wait_in(g, s):
        for d in in_descs(g, s):
            d.wait()

    def gather_descs(s):
        return [
            pltpu.make_async_copy(
                table_hbm.at[idx_v.at[s, pl.ds(j * IBLK, IBLK)]],
                rows_v.at[s, pl.ds(j * IBLK, IBLK)],
                g_sem[s],
            )
            for j in range(KBLK)
        ]

    def out_desc(g, s):
        row0 = base + g * CHUNK
        return pltpu.make_async_copy(
            rows_v.at[s], out_hbm.at[pl.ds(row0, CHUNK)], out_sem[s])

    start_in(0, 0)

    def body(i, _):
        g0 = 2 * i
        g1 = g0 + 1
        wait_in(g0, 0)
        gd0 = gather_descs(0)
        for d in gd0:
            d.start(add=True)
        @pl.when(i > 0)
        def _():
            out_desc(g0 - 1, 1).wait()
        start_in(g1, 1)
        for d in gd0:
            d.wait()
        out_desc(g0, 0).start()
        wait_in(g1, 1)
        gd1 = gather_descs(1)
        for d in gd1:
            d.start(add=True)
        out_desc(g0, 0).wait()

        @pl.when(g1 + 1 < N_CHUNKS)
        def _():
            start_in(g1 + 1, 0)
        for d in gd1:
            d.wait()
        out_desc(g1, 1).start()
        return _

    lax.fori_loop(0, N_CHUNKS // 2, body, None)
    out_desc(N_CHUNKS - 1, 1).wait()


@jax.jit
def _embed(idx_flat, token_table, pos_tiled):
    mesh = plsc.VectorSubcoreMesh(
        core_axis_name="c", subcore_axis_name="s", num_cores=NC, num_subcores=NS
    )
    fn = pl.kernel(
        _emb_kernel,
        out_type=jax.ShapeDtypeStruct((B_TOTAL, D), jnp.float32),
        mesh=mesh,
        scratch_types=[
            pltpu.VMEM((2, CHUNK), jnp.int32),
            pltpu.VMEM((2, CHUNK, D), jnp.float32),
            pltpu.SemaphoreType.DMA,
            pltpu.SemaphoreType.DMA,
            pltpu.SemaphoreType.DMA,
            pltpu.SemaphoreType.DMA,
            pltpu.SemaphoreType.DMA,
            pltpu.SemaphoreType.DMA,
        ],
        compiler_params=pltpu.CompilerParams(use_tc_tiling_on_sc=False),
    )
    return fn(idx_flat, token_table, pos_tiled)


def kernel(inputs, token_table, pos_table):
    idx_flat = inputs.astype(jnp.int32).reshape(B_TOTAL)
    pos_tiled = jnp.tile(pos_table.astype(jnp.float32), (POS_TILE // SEQ, 1))
    out = _embed(idx_flat, token_table.astype(jnp.float32), pos_tiled)
    return out.reshape(BATCH, SEQ, D)
